# baseline (speedup 1.0000x reference)
import functools

import jax
import jax.numpy as jnp
from jax import lax
from jax.experimental import pallas as pl
from jax.experimental.pallas import tpu as pltpu
from jax.experimental.pallas import tpu_sc as plsc

N = 10000
E = 160000
IN_CH = 256
TYPE_DIM = 64
D0 = IN_CH + TYPE_DIM
D0C = 384
HID = 512
OUT = 256
NUM_REL = 8

NTILES = 16
NCORES = 2
EB = 128
KBUF = 2
NB_PER_TILE = 80
E_PAD = EB * NTILES * NB_PER_TILE
NPAD = N + 16
ROWS_PER_TILE = 624

BN = 400


def _sc_aggregate(tbl, gidx, didx, d, n_chunks, zeros=None):
    W = d // n_chunks
    assert W == 128
    split = n_chunks % 2 == 1
    mesh = plsc.VectorSubcoreMesh(core_axis_name="c", subcore_axis_name="s")
    out_type = jax.ShapeDtypeStruct((N, d), jnp.float32)
    if split:
        out_type = (out_type, jax.ShapeDtypeStruct((N, W), jnp.float32))

    @functools.partial(
        pl.kernel,
        out_type=out_type,
        mesh=mesh,
        scratch_types=[
            pltpu.VMEM((NB_PER_TILE * EB,), jnp.int32),
            [pltpu.VMEM((1, EB), jnp.int32) for _ in range(KBUF)],
            [pltpu.VMEM((EB, W), jnp.float32) for _ in range(KBUF)],
            [pltpu.SemaphoreType.DMA for _ in range(KBUF)],
            [pltpu.SemaphoreType.DMA for _ in range(KBUF)],
            [pltpu.SemaphoreType.DMA for _ in range(KBUF)],
            pltpu.VMEM_SHARED((NPAD, W), jnp.float32),
        ],
    )
    def agg(tbl_hbm, gidx_hbm, didx_hbm, *rest):
        if split:
            zero_hbm, out_hbm, out2_hbm = rest[0], rest[1], rest[2]
            scr = rest[3:]
        else:
            out_hbm = rest[0]
            scr = rest[1:]
        gidx_all, didx_st, rows, sem_row, sem_idx, sem_sc, acc = scr
        cid = lax.axis_index("c")
        sid = lax.axis_index("s")
        r0 = sid * ROWS_PER_TILE
        e0 = sid * (NB_PER_TILE * EB)
        pltpu.sync_copy(gidx_hbm.at[pl.ds(e0, NB_PER_TILE * EB)], gidx_all)

        def refill(b, j, col0):
            off = pl.multiple_of(b * EB, EB)
            pltpu.async_copy(didx_hbm.at[pl.ds(e0 + off, EB)],
                             didx_st[j].at[0], sem_idx[j])
            pltpu.async_copy(
                tbl_hbm.at[gidx_all.at[pl.ds(off, EB)], pl.ds(col0, W)],
                rows[j], sem_row[j])

        def drain(j, col0):
            pltpu.make_async_copy(didx_hbm.at[pl.ds(0, EB)],
                                  didx_st[j].at[0], sem_idx[j]).wait()
            pltpu.make_async_copy(
                tbl_hbm.at[gidx_all.at[pl.ds(0, EB)], pl.ds(col0, W)],
                rows[j], sem_row[j]).wait()
            pltpu.async_copy(rows[j], acc.at[didx_st[j].at[0]], sem_sc[j],
                             add=True)

        def scatter_wait(j):
            pltpu.make_async_copy(rows[j], acc.at[didx_st[j].at[0]],
                                  sem_sc[j]).wait()

        def chunk_pass(col0, o_ref, ocol0, b0, nb, from_tbl):
            if from_tbl:
                src_main = tbl_hbm.at[pl.ds(N + r0, ROWS_PER_TILE),
                                      pl.ds(col0, W)]
                src_tail = tbl_hbm.at[pl.ds(N + 9984, 16), pl.ds(col0, W)]
            else:
                src_main = zero_hbm.at[pl.ds(r0, ROWS_PER_TILE), :]
                src_tail = zero_hbm.at[pl.ds(9984, 16), :]
            pltpu.sync_copy(src_main, acc.at[pl.ds(r0, ROWS_PER_TILE), :])

            @pl.when(sid == 0)
            def _():
                pltpu.sync_copy(src_tail, acc.at[pl.ds(9984, 16), :])

            plsc.subcore_barrier()

            for j in range(KBUF):
                refill(b0 + j, j, col0)

            def ebody(i, carry):
                for j in range(KBUF):
                    bb = i * KBUF + j
                    drain(j, col0)

                    @pl.when(bb + KBUF < nb)
                    def _(bb=bb, j=j):
                        scatter_wait(j)
                        refill(b0 + bb + KBUF, j, col0)
                return carry

            lax.fori_loop(0, nb // KBUF, ebody, 0)
            for j in range(KBUF):
                scatter_wait(j)
            plsc.subcore_barrier()
            pltpu.sync_copy(
                acc.at[pl.ds(r0, ROWS_PER_TILE), :],
                o_ref.at[pl.ds(r0, ROWS_PER_TILE), pl.ds(ocol0, W)],
            )

            @pl.when(sid == 0)
            def _():
                pltpu.sync_copy(
                    acc.at[pl.ds(9984, 16), :],
                    o_ref.at[pl.ds(9984, 16), pl.ds(ocol0, W)],
                )

        nfull = n_chunks - 1 if split else n_chunks
        cpc = nfull // NCORES
        for cc in range(nfull):
            @pl.when(cid == cc // cpc)
            def _(cc=cc):
                chunk_pass(cc * W, out_hbm, cc * W, 0, NB_PER_TILE, True)
        if split:
            nb0 = 76
            lastc = (n_chunks - 1) * W

            @pl.when(cid == 0)
            def _():
                chunk_pass(lastc, out_hbm, lastc, 0, nb0, True)

            @pl.when(cid == 1)
            def _():
                chunk_pass(lastc, out2_hbm, 0, nb0, NB_PER_TILE - nb0, False)

    if split:
        return agg(tbl, gidx, didx, zeros)
    return agg(tbl, gidx, didx)


def _tc_prep(x, ids2d, temb, rel1):
    def body(x_ref, ids_ref, temb_ref, rel_ref, h0_ref, t_ref):
        xb = x_ref[...]
        ids = ids_ref[...]
        oh = (ids == lax.broadcasted_iota(jnp.int32, (BN, NUM_REL), 1))
        tp = jnp.dot(oh.astype(jnp.float32), temb_ref[...],
                     preferred_element_type=jnp.float32)
        h0 = jnp.concatenate([xb, tp], axis=1)
        h0_ref[...] = h0
        t = h0[None] * rel_ref[...][:, None, :]
        ones = jnp.ones((NUM_REL, BN, D0C - D0), jnp.float32)
        t_ref[...] = jnp.concatenate([t, ones], axis=2)

    return pl.pallas_call(
        body,
        grid=(N // BN,),
        in_specs=[
            pl.BlockSpec((BN, IN_CH), lambda i: (i, 0)),
            pl.BlockSpec((BN, 1), lambda i: (i, 0)),
            pl.BlockSpec((NUM_REL, TYPE_DIM), lambda i: (0, 0)),
            pl.BlockSpec((NUM_REL, D0), lambda i: (0, 0)),
        ],
        out_specs=[
            pl.BlockSpec((BN, D0), lambda i: (i, 0)),
            pl.BlockSpec((NUM_REL, BN, D0C), lambda i: (0, i, 0)),
        ],
        out_shape=[
            jax.ShapeDtypeStruct((N, D0), jnp.float32),
            jax.ShapeDtypeStruct((NUM_REL, N, D0C), jnp.float32),
        ],
    )(x, ids2d, temb, rel1)


def _tc_update1(s1, s1b, h0, wo1, wl1, rel2):
    def body(s_ref, sb_ref, h_ref, wo_ref, wl_ref, rel_ref,
             h1_ref, t_ref, inv_ref):
        s = s_ref[...]
        c2 = s[:, 256:D0C] + sb_ref[...]
        inv = 1.0 / c2[:, 64:65]
        aggr = jnp.concatenate([s[:, :256], c2[:, :64]], axis=1) * inv
        out = (jnp.dot(aggr, wo_ref[...], preferred_element_type=jnp.float32)
               + jnp.dot(h_ref[...], wl_ref[...],
                         preferred_element_type=jnp.float32))
        h1 = jnp.maximum(out, 0.0)
        h1_ref[...] = h1
        t_ref[...] = h1[None] * rel_ref[...][:, None, :]
        inv_ref[...] = jnp.broadcast_to(inv, (BN, 16))

    return pl.pallas_call(
        body,
        grid=(N // BN,),
        in_specs=[
            pl.BlockSpec((BN, D0C), lambda i: (i, 0)),
            pl.BlockSpec((BN, 128), lambda i: (i, 0)),
            pl.BlockSpec((BN, D0), lambda i: (i, 0)),
            pl.BlockSpec((D0, HID), lambda i: (0, 0)),
            pl.BlockSpec((D0, HID), lambda i: (0, 0)),
            pl.BlockSpec((NUM_REL, HID), lambda i: (0, 0)),
        ],
        out_specs=[
            pl.BlockSpec((BN, HID), lambda i: (i, 0)),
            pl.BlockSpec((NUM_REL, BN, HID), lambda i: (0, i, 0)),
            pl.BlockSpec((BN, 16), lambda i: (i, 0)),
        ],
        out_shape=[
            jax.ShapeDtypeStruct((N, HID), jnp.float32),
            jax.ShapeDtypeStruct((NUM_REL, N, HID), jnp.float32),
            jax.ShapeDtypeStruct((N, 16), jnp.float32),
        ],
    )(s1, s1b, h0, wo1, wl1, rel2)


def _tc_update2(s, inv16, h, wo, wl, rel3, wo3):
    def body(s_ref, inv_ref, h_ref, wo_ref, wl_ref, rel_ref, wo3_ref,
             h2_ref, g3_ref):
        aggr = s_ref[...] * inv_ref[...][:, 0:1]
        out = (jnp.dot(aggr, wo_ref[...], preferred_element_type=jnp.float32)
               + jnp.dot(h_ref[...], wl_ref[...],
                         preferred_element_type=jnp.float32))
        h2 = jnp.maximum(out, 0.0)
        h2_ref[...] = h2
        wo3 = wo3_ref[...]
        rel = rel_ref[...]
        for r in range(NUM_REL):
            g3_ref[r] = jnp.dot(h2, rel[r][:, None] * wo3,
                                preferred_element_type=jnp.float32)

    return pl.pallas_call(
        body,
        grid=(N // BN,),
        in_specs=[
            pl.BlockSpec((BN, HID), lambda i: (i, 0)),
            pl.BlockSpec((BN, 16), lambda i: (i, 0)),
            pl.BlockSpec((BN, HID), lambda i: (i, 0)),
            pl.BlockSpec((HID, HID), lambda i: (0, 0)),
            pl.BlockSpec((HID, HID), lambda i: (0, 0)),
            pl.BlockSpec((NUM_REL, HID), lambda i: (0, 0)),
            pl.BlockSpec((HID, OUT), lambda i: (0, 0)),
        ],
        out_specs=[
            pl.BlockSpec((BN, HID), lambda i: (i, 0)),
            pl.BlockSpec((NUM_REL, BN, OUT), lambda i: (0, i, 0)),
        ],
        out_shape=[
            jax.ShapeDtypeStruct((N, HID), jnp.float32),
            jax.ShapeDtypeStruct((NUM_REL, N, OUT), jnp.float32),
        ],
    )(s, inv16, h, wo, wl, rel3, wo3)


def _tc_update3(s, inv16, h, wl):
    def body(s_ref, inv_ref, h_ref, wl_ref, o_ref):
        o_ref[...] = (
            s_ref[...] * inv_ref[...][:, 0:1]
            + jnp.dot(h_ref[...], wl_ref[...],
                      preferred_element_type=jnp.float32))

    return pl.pallas_call(
        body,
        grid=(N // BN,),
        in_specs=[
            pl.BlockSpec((BN, OUT), lambda i: (i, 0)),
            pl.BlockSpec((BN, 16), lambda i: (i, 0)),
            pl.BlockSpec((BN, HID), lambda i: (i, 0)),
            pl.BlockSpec((HID, OUT), lambda i: (0, 0)),
        ],
        out_specs=pl.BlockSpec((BN, OUT), lambda i: (i, 0)),
        out_shape=jax.ShapeDtypeStruct((N, OUT), jnp.float32),
    )(s, inv16, h, wl)


def kernel(x, node_type_ids, edge_index, edge_type, type_emb,
           rel1, wl1, wo1, rel2, wl2, wo2, rel3, wl3, wo3):
    src = edge_index[0].astype(jnp.int32)
    dst = edge_index[1].astype(jnp.int32)
    et = edge_type.astype(jnp.int32)
    gidx = et * N + src
    pad = E_PAD - E
    gidx_p = jnp.concatenate([gidx, jnp.zeros((pad,), jnp.int32)])
    didx_p = jnp.concatenate([dst, jnp.full((pad,), N, jnp.int32)])
    ids2d = node_type_ids.astype(jnp.int32).reshape(N, 1)

    zeros128 = jnp.zeros((N, 128), jnp.float32)

    h0, t1 = _tc_prep(x, ids2d, type_emb, rel1)
    s1, s1b = _sc_aggregate(t1.reshape(NUM_REL * N, D0C), gidx_p, didx_p,
                            D0C, 3, zeros=zeros128)
    h1, t2, inv16 = _tc_update1(s1, s1b, h0, wo1, wl1, rel2)
    s2 = _sc_aggregate(t2.reshape(NUM_REL * N, HID), gidx_p, didx_p, HID, 4)
    h2, g3 = _tc_update2(s2, inv16, h1, wo2, wl2, rel3, wo3)
    s3 = _sc_aggregate(g3.reshape(NUM_REL * N, OUT), gidx_p, didx_p, OUT, 2)
    return _tc_update3(s3, inv16, h2, wl3)


# speedup vs baseline: 1.0340x; 1.0340x over previous
import functools

import jax
import jax.numpy as jnp
from jax import lax
from jax.experimental import pallas as pl
from jax.experimental.pallas import tpu as pltpu
from jax.experimental.pallas import tpu_sc as plsc

N = 10000
E = 160000
IN_CH = 256
TYPE_DIM = 64
D0 = IN_CH + TYPE_DIM
D0C = 384
HID = 512
OUT = 256
NUM_REL = 8

NTILES = 16
NCORES = 2
EB = 128
KBUF = 2
NB_PER_TILE = 80
E_PAD = EB * NTILES * NB_PER_TILE
NPAD = N + 16
ROWS_PER_TILE = 624

BN = 400


def _sc_aggregate(tbl, gidx, didx, d, n_chunks, zeros=None):
    W = d // n_chunks
    assert W == 128
    split = n_chunks % 2 == 1
    mesh = plsc.VectorSubcoreMesh(core_axis_name="c", subcore_axis_name="s")
    out_type = jax.ShapeDtypeStruct((N, d), jnp.float32)
    if split:
        out_type = (out_type, jax.ShapeDtypeStruct((N, W), jnp.float32))

    @functools.partial(
        pl.kernel,
        out_type=out_type,
        mesh=mesh,
        scratch_types=[
            pltpu.VMEM((NB_PER_TILE * EB,), jnp.int32),
            [pltpu.VMEM((1, EB), jnp.int32) for _ in range(KBUF)],
            [pltpu.VMEM((EB, W), jnp.float32) for _ in range(KBUF)],
            [pltpu.SemaphoreType.DMA for _ in range(KBUF)],
            [pltpu.SemaphoreType.DMA for _ in range(KBUF)],
            [pltpu.SemaphoreType.DMA for _ in range(KBUF)],
            pltpu.VMEM_SHARED((NPAD, W), jnp.float32),
        ],
    )
    def agg(tbl_hbm, gidx_hbm, didx_hbm, *rest):
        if split:
            zero_hbm, out_hbm, out2_hbm = rest[0], rest[1], rest[2]
            scr = rest[3:]
        else:
            out_hbm = rest[0]
            scr = rest[1:]
        gidx_all, didx_st, rows, sem_row, sem_idx, sem_sc, acc = scr
        cid = lax.axis_index("c")
        sid = lax.axis_index("s")
        r0 = sid * ROWS_PER_TILE
        e0 = sid * (NB_PER_TILE * EB)
        pltpu.sync_copy(gidx_hbm.at[pl.ds(e0, NB_PER_TILE * EB)], gidx_all)

        def refill(b, j, col0):
            off = pl.multiple_of(b * EB, EB)
            pltpu.async_copy(didx_hbm.at[pl.ds(e0 + off, EB)],
                             didx_st[j].at[0], sem_idx[j])
            pltpu.async_copy(
                tbl_hbm.at[gidx_all.at[pl.ds(off, EB)], pl.ds(col0, W)],
                rows[j], sem_row[j])

        def drain(j, col0):
            pltpu.make_async_copy(didx_hbm.at[pl.ds(0, EB)],
                                  didx_st[j].at[0], sem_idx[j]).wait()
            pltpu.make_async_copy(
                tbl_hbm.at[gidx_all.at[pl.ds(0, EB)], pl.ds(col0, W)],
                rows[j], sem_row[j]).wait()
            pltpu.async_copy(rows[j], acc.at[didx_st[j].at[0]], sem_sc[j],
                             add=True)

        def scatter_wait(j):
            pltpu.make_async_copy(rows[j], acc.at[didx_st[j].at[0]],
                                  sem_sc[j]).wait()

        def chunk_pass(col0, o_ref, ocol0, b0, nb, from_tbl):
            if from_tbl:
                src_main = tbl_hbm.at[pl.ds(N + r0, ROWS_PER_TILE),
                                      pl.ds(col0, W)]
                src_tail = tbl_hbm.at[pl.ds(N + 9984, 16), pl.ds(col0, W)]
            else:
                src_main = zero_hbm.at[pl.ds(r0, ROWS_PER_TILE), :]
                src_tail = zero_hbm.at[pl.ds(9984, 16), :]
            pltpu.sync_copy(src_main, acc.at[pl.ds(r0, ROWS_PER_TILE), :])

            @pl.when(sid == 0)
            def _():
                pltpu.sync_copy(src_tail, acc.at[pl.ds(9984, 16), :])

            plsc.subcore_barrier()

            for j in range(KBUF):
                refill(b0 + j, j, col0)

            def ebody(i, carry):
                for j in range(KBUF):
                    bb = i * KBUF + j
                    drain(j, col0)

                    @pl.when(bb + KBUF < nb)
                    def _(bb=bb, j=j):
                        scatter_wait(j)
                        refill(b0 + bb + KBUF, j, col0)
                return carry

            lax.fori_loop(0, nb // KBUF, ebody, 0)
            for j in range(KBUF):
                scatter_wait(j)
            plsc.subcore_barrier()
            pltpu.sync_copy(
                acc.at[pl.ds(r0, ROWS_PER_TILE), :],
                o_ref.at[pl.ds(r0, ROWS_PER_TILE), pl.ds(ocol0, W)],
            )

            @pl.when(sid == 0)
            def _():
                pltpu.sync_copy(
                    acc.at[pl.ds(9984, 16), :],
                    o_ref.at[pl.ds(9984, 16), pl.ds(ocol0, W)],
                )

        nfull = n_chunks - 1 if split else n_chunks
        cpc = nfull // NCORES
        for cc in range(nfull):
            @pl.when(cid == cc // cpc)
            def _(cc=cc):
                chunk_pass(cc * W, out_hbm, cc * W, 0, NB_PER_TILE, True)
        if split:
            nb0 = 60
            lastc = (n_chunks - 1) * W

            @pl.when(cid == 0)
            def _():
                chunk_pass(lastc, out_hbm, lastc, 0, nb0, True)

            @pl.when(cid == 1)
            def _():
                chunk_pass(lastc, out2_hbm, 0, nb0, NB_PER_TILE - nb0, False)

    if split:
        return agg(tbl, gidx, didx, zeros)
    return agg(tbl, gidx, didx)


def _tc_prep(x, ids2d, temb, rel1):
    def body(x_ref, ids_ref, temb_ref, rel_ref, h0_ref, t_ref):
        xb = x_ref[...]
        ids = ids_ref[...]
        oh = (ids == lax.broadcasted_iota(jnp.int32, (BN, NUM_REL), 1))
        tp = jnp.dot(oh.astype(jnp.float32), temb_ref[...],
                     preferred_element_type=jnp.float32)
        h0 = jnp.concatenate([xb, tp], axis=1)
        h0_ref[...] = h0
        t = h0[None] * rel_ref[...][:, None, :]
        ones = jnp.ones((NUM_REL, BN, D0C - D0), jnp.float32)
        t_ref[...] = jnp.concatenate([t, ones], axis=2)

    return pl.pallas_call(
        body,
        grid=(N // BN,),
        in_specs=[
            pl.BlockSpec((BN, IN_CH), lambda i: (i, 0)),
            pl.BlockSpec((BN, 1), lambda i: (i, 0)),
            pl.BlockSpec((NUM_REL, TYPE_DIM), lambda i: (0, 0)),
            pl.BlockSpec((NUM_REL, D0), lambda i: (0, 0)),
        ],
        out_specs=[
            pl.BlockSpec((BN, D0), lambda i: (i, 0)),
            pl.BlockSpec((NUM_REL, BN, D0C), lambda i: (0, i, 0)),
        ],
        out_shape=[
            jax.ShapeDtypeStruct((N, D0), jnp.float32),
            jax.ShapeDtypeStruct((NUM_REL, N, D0C), jnp.float32),
        ],
    )(x, ids2d, temb, rel1)


def _tc_update1(s1, s1b, h0, wo1, wl1, rel2):
    def body(s_ref, sb_ref, h_ref, wo_ref, wl_ref, rel_ref,
             h1_ref, t_ref, inv_ref):
        s = s_ref[...]
        c2 = s[:, 256:D0C] + sb_ref[...]
        inv = 1.0 / c2[:, 64:65]
        aggr = jnp.concatenate([s[:, :256], c2[:, :64]], axis=1) * inv
        out = (jnp.dot(aggr, wo_ref[...], preferred_element_type=jnp.float32)
               + jnp.dot(h_ref[...], wl_ref[...],
                         preferred_element_type=jnp.float32))
        h1 = jnp.maximum(out, 0.0)
        h1_ref[...] = h1
        t_ref[...] = h1[None] * rel_ref[...][:, None, :]
        inv_ref[...] = jnp.broadcast_to(inv, (BN, 16))

    return pl.pallas_call(
        body,
        grid=(N // BN,),
        in_specs=[
            pl.BlockSpec((BN, D0C), lambda i: (i, 0)),
            pl.BlockSpec((BN, 128), lambda i: (i, 0)),
            pl.BlockSpec((BN, D0), lambda i: (i, 0)),
            pl.BlockSpec((D0, HID), lambda i: (0, 0)),
            pl.BlockSpec((D0, HID), lambda i: (0, 0)),
            pl.BlockSpec((NUM_REL, HID), lambda i: (0, 0)),
        ],
        out_specs=[
            pl.BlockSpec((BN, HID), lambda i: (i, 0)),
            pl.BlockSpec((NUM_REL, BN, HID), lambda i: (0, i, 0)),
            pl.BlockSpec((BN, 16), lambda i: (i, 0)),
        ],
        out_shape=[
            jax.ShapeDtypeStruct((N, HID), jnp.float32),
            jax.ShapeDtypeStruct((NUM_REL, N, HID), jnp.float32),
            jax.ShapeDtypeStruct((N, 16), jnp.float32),
        ],
    )(s1, s1b, h0, wo1, wl1, rel2)


def _tc_update2(s, inv16, h, wo, wl, rel3, wo3):
    def body(s_ref, inv_ref, h_ref, wo_ref, wl_ref, rel_ref, wo3_ref,
             h2_ref, g3_ref):
        aggr = s_ref[...] * inv_ref[...][:, 0:1]
        out = (jnp.dot(aggr, wo_ref[...], preferred_element_type=jnp.float32)
               + jnp.dot(h_ref[...], wl_ref[...],
                         preferred_element_type=jnp.float32))
        h2 = jnp.maximum(out, 0.0)
        h2_ref[...] = h2
        wo3 = wo3_ref[...]
        rel = rel_ref[...]
        for r in range(NUM_REL):
            g3_ref[r] = jnp.dot(h2, rel[r][:, None] * wo3,
                                preferred_element_type=jnp.float32)

    return pl.pallas_call(
        body,
        grid=(N // BN,),
        in_specs=[
            pl.BlockSpec((BN, HID), lambda i: (i, 0)),
            pl.BlockSpec((BN, 16), lambda i: (i, 0)),
            pl.BlockSpec((BN, HID), lambda i: (i, 0)),
            pl.BlockSpec((HID, HID), lambda i: (0, 0)),
            pl.BlockSpec((HID, HID), lambda i: (0, 0)),
            pl.BlockSpec((NUM_REL, HID), lambda i: (0, 0)),
            pl.BlockSpec((HID, OUT), lambda i: (0, 0)),
        ],
        out_specs=[
            pl.BlockSpec((BN, HID), lambda i: (i, 0)),
            pl.BlockSpec((NUM_REL, BN, OUT), lambda i: (0, i, 0)),
        ],
        out_shape=[
            jax.ShapeDtypeStruct((N, HID), jnp.float32),
            jax.ShapeDtypeStruct((NUM_REL, N, OUT), jnp.float32),
        ],
    )(s, inv16, h, wo, wl, rel3, wo3)


def _tc_update3(s, inv16, h, wl):
    def body(s_ref, inv_ref, h_ref, wl_ref, o_ref):
        o_ref[...] = (
            s_ref[...] * inv_ref[...][:, 0:1]
            + jnp.dot(h_ref[...], wl_ref[...],
                      preferred_element_type=jnp.float32))

    return pl.pallas_call(
        body,
        grid=(N // BN,),
        in_specs=[
            pl.BlockSpec((BN, OUT), lambda i: (i, 0)),
            pl.BlockSpec((BN, 16), lambda i: (i, 0)),
            pl.BlockSpec((BN, HID), lambda i: (i, 0)),
            pl.BlockSpec((HID, OUT), lambda i: (0, 0)),
        ],
        out_specs=pl.BlockSpec((BN, OUT), lambda i: (i, 0)),
        out_shape=jax.ShapeDtypeStruct((N, OUT), jnp.float32),
    )(s, inv16, h, wl)


def kernel(x, node_type_ids, edge_index, edge_type, type_emb,
           rel1, wl1, wo1, rel2, wl2, wo2, rel3, wl3, wo3):
    src = edge_index[0].astype(jnp.int32)
    dst = edge_index[1].astype(jnp.int32)
    et = edge_type.astype(jnp.int32)
    gidx = et * N + src
    pad = E_PAD - E
    gidx_p = jnp.concatenate([gidx, jnp.zeros((pad,), jnp.int32)])
    didx_p = jnp.concatenate([dst, jnp.full((pad,), N, jnp.int32)])
    ids2d = node_type_ids.astype(jnp.int32).reshape(N, 1)

    zeros128 = jnp.zeros((N, 128), jnp.float32)

    h0, t1 = _tc_prep(x, ids2d, type_emb, rel1)
    s1, s1b = _sc_aggregate(t1.reshape(NUM_REL * N, D0C), gidx_p, didx_p,
                            D0C, 3, zeros=zeros128)
    h1, t2, inv16 = _tc_update1(s1, s1b, h0, wo1, wl1, rel2)
    s2 = _sc_aggregate(t2.reshape(NUM_REL * N, HID), gidx_p, didx_p, HID, 4)
    h2, g3 = _tc_update2(s2, inv16, h1, wo2, wl2, rel3, wo3)
    s3 = _sc_aggregate(g3.reshape(NUM_REL * N, OUT), gidx_p, didx_p, OUT, 2)
    return _tc_update3(s3, inv16, h2, wl3)
